# initial kernel scaffold (unmeasured)
import jax
import jax.numpy as jnp
from jax import lax
from jax.experimental import pallas as pl
from jax.experimental.pallas import tpu as pltpu

N_DEV = 4
N_PANELS = 4


def kernel(x, w_mat):
    x = x.astype(jnp.bfloat16)
    w_mat = w_mat.astype(jnp.bfloat16)
    m_global, k_per = x.shape
    _, n = w_mat.shape
    m_per = m_global // N_DEV
    panel = n // N_PANELS

    def body(x_hbm, w_ref, out_ref, xbuf, comm, send_sems, recv_sems,
             xsem, credit_sem):
        my = lax.axis_index("i")
        left = (my - 1) % N_DEV
        right = (my + 1) % N_DEV

        barrier_sem = pltpu.get_barrier_semaphore()
        for nbr in (left, right):
            pl.semaphore_signal(
                barrier_sem, inc=1,
                device_id=(nbr,), device_id_type=pl.DeviceIdType.MESH,
            )
        pl.semaphore_wait(barrier_sem, 2)

        def load_chunk(c):
            cp = pltpu.make_async_copy(
                x_hbm.at[pl.ds(c * m_per, m_per), :], xbuf, xsem)
            cp.start()
            cp.wait()

        def partial_panel(p):
            wp = w_ref[:, p * panel:(p + 1) * panel]
            return jnp.dot(xbuf[...], wp, preferred_element_type=jnp.float32)

        load_chunk((my - 1) % N_DEV)
        for p in range(N_PANELS):
            comm[0, :, p * panel:(p + 1) * panel] = (
                partial_panel(p).astype(jnp.bfloat16))

        for h in range(N_DEV - 1):
            send_slot = h % 2
            recv_slot = (h + 1) % 2
            if h >= 2:
                pl.semaphore_wait(credit_sem, 1)
            rdma = pltpu.make_async_remote_copy(
                src_ref=comm.at[send_slot],
                dst_ref=comm.at[recv_slot],
                send_sem=send_sems.at[h],
                recv_sem=recv_sems.at[h],
                device_id=(right,),
                device_id_type=pl.DeviceIdType.MESH,
            )
            rdma.start()
            load_chunk((my - h - 2) % N_DEV)
            rdma.wait()
            if h == 0:
                pl.semaphore_signal(
                    credit_sem, inc=1,
                    device_id=(left,), device_id_type=pl.DeviceIdType.MESH,
                )
            if h < N_DEV - 2:
                for p in range(N_PANELS):
                    s = (comm[recv_slot, :, p * panel:(p + 1) * panel]
                         .astype(jnp.float32) + partial_panel(p))
                    comm[recv_slot, :, p * panel:(p + 1) * panel] = (
                        s.astype(jnp.bfloat16))
            else:
                for p in range(N_PANELS):
                    s = (comm[recv_slot, :, p * panel:(p + 1) * panel]
                         .astype(jnp.float32) + partial_panel(p))
                    s = jnp.clip(s, -60.0, 60.0)
                    out_ref[:, p * panel:(p + 1) * panel] = (
                        s / (1.0 + jnp.exp(-s)))

    return pl.pallas_call(
        body,
        out_shape=jax.ShapeDtypeStruct((m_per, n), jnp.float32),
        in_specs=[
            pl.BlockSpec(memory_space=pltpu.ANY),
            pl.BlockSpec(memory_space=pltpu.VMEM),
        ],
        out_specs=pl.BlockSpec(memory_space=pltpu.VMEM),
        scratch_shapes=[
            pltpu.VMEM((m_per, k_per), jnp.bfloat16),
            pltpu.VMEM((2, m_per, n), jnp.bfloat16),
            pltpu.SemaphoreType.DMA((N_DEV - 1,)),
            pltpu.SemaphoreType.DMA((N_DEV - 1,)),
            pltpu.SemaphoreType.DMA,
            pltpu.SemaphoreType.REGULAR,
        ],
        compiler_params=pltpu.CompilerParams(
            collective_id=0,
            vmem_limit_bytes=128 * 1024 * 1024,
        ),
    )(x, w_mat)


# baseline (device time: 868935 ns/iter reference)
import jax
import jax.numpy as jnp
from jax import lax
from jax.experimental import pallas as pl
from jax.experimental.pallas import tpu as pltpu

N_DEV = 4
N_HOPS = N_DEV - 1
BLK = 512
N_BLOCKS = 4096 // BLK
G_TOT = N_BLOCKS * N_HOPS


def kernel(x, w_mat):
    x = x.astype(jnp.bfloat16)
    w_mat = w_mat.astype(jnp.bfloat16)
    m_global, k_per = x.shape
    _, n = w_mat.shape
    m_per = m_global // N_DEV

    def body(x_hbm, w_ref, out_ref, xbuf, comm, send_sems, recv_sems,
             xsem, credit_sem):
        j = pl.program_id(0)
        my = lax.axis_index("i")
        left = (my - 1) % N_DEV
        right = (my + 1) % N_DEV

        @pl.when(j == 0)
        def _():
            barrier_sem = pltpu.get_barrier_semaphore()
            for nbr in (left, right):
                pl.semaphore_signal(
                    barrier_sem, inc=1,
                    device_id=(nbr,), device_id_type=pl.DeviceIdType.MESH,
                )
            pl.semaphore_wait(barrier_sem, 2)

        def load_chunk(c):
            cp = pltpu.make_async_copy(
                x_hbm.at[pl.ds(c * m_per, m_per), :], xbuf, xsem)
            cp.start()
            cp.wait()

        def partial():
            return jnp.dot(xbuf[...], w_ref[...],
                           preferred_element_type=jnp.float32)

        load_chunk((my - 1) % N_DEV)
        comm[(j + 0) % 2] = partial().astype(jnp.bfloat16)

        for h in range(N_HOPS):
            send_slot = (j + h) % 2
            recv_slot = (j + h + 1) % 2

            def _credit_wait():
                pl.semaphore_wait(credit_sem, 1)
            if h == 2:
                _credit_wait()
            else:
                pl.when(j >= 1)(_credit_wait)

            rdma = pltpu.make_async_remote_copy(
                src_ref=comm.at[send_slot],
                dst_ref=comm.at[recv_slot],
                send_sem=send_sems.at[h],
                recv_sem=recv_sems.at[h],
                device_id=(right,),
                device_id_type=pl.DeviceIdType.MESH,
            )
            rdma.start()
            load_chunk((my - h - 2) % N_DEV)
            rdma.wait()

            def _credit_signal():
                pl.semaphore_signal(
                    credit_sem, inc=1,
                    device_id=(left,), device_id_type=pl.DeviceIdType.MESH,
                )
            if h == 0:
                pl.when(j >= 1)(_credit_signal)
            elif h == 1:
                _credit_signal()
            else:
                pl.when(j <= N_BLOCKS - 2)(_credit_signal)

            s = comm[recv_slot].astype(jnp.float32) + partial()
            if h < N_HOPS - 1:
                comm[recv_slot] = s.astype(jnp.bfloat16)
            else:
                out_ref[...] = s / (1.0 + jnp.exp(-jnp.clip(s, -60.0, 60.0)))

    return pl.pallas_call(
        body,
        grid=(N_BLOCKS,),
        out_shape=jax.ShapeDtypeStruct((m_per, n), jnp.float32),
        in_specs=[
            pl.BlockSpec(memory_space=pl.ANY),
            pl.BlockSpec((k_per, BLK), lambda j: (0, j)),
        ],
        out_specs=pl.BlockSpec((m_per, BLK), lambda j: (0, j)),
        scratch_shapes=[
            pltpu.VMEM((m_per, k_per), jnp.bfloat16),
            pltpu.VMEM((2, m_per, BLK), jnp.bfloat16),
            pltpu.SemaphoreType.DMA((N_HOPS,)),
            pltpu.SemaphoreType.DMA((N_HOPS,)),
            pltpu.SemaphoreType.DMA,
            pltpu.SemaphoreType.REGULAR,
        ],
        compiler_params=pltpu.CompilerParams(
            collective_id=0,
            dimension_semantics=("arbitrary",),
            vmem_limit_bytes=64 * 1024 * 1024,
        ),
    )(x, w_mat)
